# baseline (speedup 1.0000x reference)
import functools
import numpy as np
import jax
import jax.numpy as jnp
from jax import lax
from jax.experimental import pallas as pl
from jax.experimental.pallas import tpu as pltpu
from jax.experimental.pallas import tpu_sc as plsc

_N_NODES = 10000
_D = 32
_NB = 10
_H = 16
_CUTOFF = 4.0
_NG = 64
_SH0 = 0.28209479177387814
_SILU_2MOM = 1.6790590286254883
_PATH_W = 1.0 / float(np.sqrt(_D))
_MSG_SCALE = _SH0 * _PATH_W

_NW = 32
_E_PAD = 163840
_ECH = 128
_ENCH = _E_PAD // (_NW * _ECH)
_SNCH = _E_PAD // (16 * _ECH)
_N_PAD = 10240
_ZCH = 64
_ZNCH = _N_PAD // (_NW * _ZCH)
_NHALF = 5000
_NH = 5120
_BE = 2048

_SC_PARAMS = pltpu.CompilerParams(use_tc_tiling_on_sc=False)


def _sc_mesh():
    return plsc.VectorSubcoreMesh(core_axis_name="c", subcore_axis_name="s")


_GRP = 4


def _sc_gather0(pos16, src3, tgt3):
    @functools.partial(
        pl.kernel,
        out_type=(jax.ShapeDtypeStruct((_E_PAD, 16), jnp.float32),
                  jax.ShapeDtypeStruct((_E_PAD, 16), jnp.float32)),
        mesh=_sc_mesh(),
        compiler_params=_SC_PARAMS,
        scratch_types=[
            pltpu.VMEM((_ENCH, _ECH), jnp.int32),
            pltpu.VMEM((_GRP * _ECH, 16), jnp.float32),
            pltpu.SemaphoreType.DMA,
        ],
    )
    def k(pos_hbm, src_hbm, tgt_hbm, ps_out, pt_out, eidx, ebuf, sem):
        wid = lax.axis_index("s") * 2 + lax.axis_index("c")
        base = wid * (_ENCH * _ECH)
        for idx_hbm, out in ((src_hbm, ps_out), (tgt_hbm, pt_out)):
            pltpu.sync_copy(idx_hbm.at[wid], eidx)

            def body(g, carry, out=out):
                cps = [
                    pltpu.async_copy(
                        pos_hbm.at[eidx.at[g * _GRP + j]],
                        ebuf.at[pl.ds(j * _ECH, _ECH)], sem)
                    for j in range(_GRP)
                ]
                for cp in cps:
                    cp.wait()
                pltpu.sync_copy(
                    ebuf, out.at[pl.ds(base + g * (_GRP * _ECH),
                                       _GRP * _ECH)])
                return carry
            lax.fori_loop(0, _ENCH // _GRP, body, 0)

    return k(pos16, src3, tgt3)


def _sc_gather(table, idx3):
    nw, nch, ch = idx3.shape
    out_rows = nw * nch * ch

    @functools.partial(
        pl.kernel,
        out_type=jax.ShapeDtypeStruct((out_rows, _D), jnp.float32),
        mesh=_sc_mesh(),
        compiler_params=_SC_PARAMS,
        scratch_types=[
            pltpu.VMEM((nch, ch), jnp.int32),
            pltpu.VMEM((_GRP * ch, _D), jnp.float32),
            pltpu.SemaphoreType.DMA,
        ],
    )
    def k(table_hbm, idx_hbm, out_hbm, idx_v, buf, sem):
        wid = lax.axis_index("s") * 2 + lax.axis_index("c")
        base = wid * (nch * ch)
        pltpu.sync_copy(idx_hbm.at[wid], idx_v)

        def body(g, carry):
            cps = [
                pltpu.async_copy(table_hbm.at[idx_v.at[g * _GRP + j]],
                                 buf.at[pl.ds(j * ch, ch)], sem)
                for j in range(_GRP)
            ]
            for cp in cps:
                cp.wait()
            pltpu.sync_copy(buf, out_hbm.at[pl.ds(base + g * (_GRP * ch),
                                                  _GRP * ch)])
            return carry
        lax.fori_loop(0, nch // _GRP, body, 0)

    return k(table, idx3)


def _sc_scatter_add(msgs, idx3, zeros, nh, nreal):
    zrows = nh // 16
    wrows = nreal // 16

    @functools.partial(
        pl.kernel,
        out_type=jax.ShapeDtypeStruct((2 * nreal, _D), jnp.float32),
        mesh=_sc_mesh(),
        compiler_params=_SC_PARAMS,
        scratch_types=[
            pltpu.VMEM((_SNCH, _ECH), jnp.int32),
            pltpu.VMEM((_GRP * _ECH, _D), jnp.float32),
            pltpu.VMEM_SHARED((nh, _D), jnp.float32),
            pltpu.SemaphoreType.DMA,
        ],
    )
    def k(msgs_hbm, idx_hbm, zeros_hbm, out_hbm, idx_v, buf, acc, sem):
        cid = lax.axis_index("c")
        sid = lax.axis_index("s")
        pltpu.sync_copy(zeros_hbm.at[pl.ds(sid * zrows, zrows)],
                        acc.at[pl.ds(sid * zrows, zrows)])
        plsc.subcore_barrier()
        pltpu.sync_copy(idx_hbm.at[cid * 16 + sid], idx_v)

        def body(g, carry):
            rows = (sid * _SNCH + g * _GRP) * _ECH
            pltpu.async_copy(msgs_hbm.at[pl.ds(rows, _GRP * _ECH)],
                             buf, sem).wait()
            for j in range(_GRP):
                pltpu.sync_copy(buf.at[pl.ds(j * _ECH, _ECH)],
                                acc.at[idx_v.at[g * _GRP + j]], add=True)
            return carry
        lax.fori_loop(0, _SNCH // _GRP, body, 0)
        plsc.subcore_barrier()
        pltpu.sync_copy(acc.at[pl.ds(sid * wrows, wrows)],
                        out_hbm.at[pl.ds(cid * nreal + sid * wrows, wrows)])

    return k(msgs, idx3, zeros)


def _bessel(ps, pt):
    rel = pt - ps
    d2 = jnp.sum(rel * rel, axis=1, keepdims=True)
    valid = (d2 > 0.0).astype(jnp.float32)
    xx = jnp.sqrt(d2) * (1.0 / _CUTOFF)
    inv = jnp.where(xx > 0.0, 1.0 / jnp.maximum(xx, 1e-30), 0.0)
    x5 = xx * xx * xx * xx * xx
    env = (inv - 28.0 * x5 + 48.0 * x5 * xx - 21.0 * x5 * xx * xx)
    env = env * (xx < 1.0).astype(jnp.float32)
    freq = np.pi * (lax.broadcasted_iota(jnp.int32, (1, _NB), 1)
                    .astype(jnp.float32) + 1.0)
    return env * jnp.sin(freq * xx), valid


def _conv_math(rb, vm, xin, w1_ref, b1_ref, r_ref, t_ref, w2z_ref, b2m_ref):
    h = jnp.dot(rb, w1_ref[...],
                preferred_element_type=jnp.float32) + b1_ref[...]
    h = h * jax.nn.sigmoid(h)
    hz = jnp.dot(h, r_ref[...], preferred_element_type=jnp.float32)
    xz = jnp.dot(xin, t_ref[...], preferred_element_type=jnp.float32)
    msg = jnp.dot(hz * xz, w2z_ref[...], preferred_element_type=jnp.float32)
    msg = msg + jnp.dot(xin, b2m_ref[...], preferred_element_type=jnp.float32)
    return (_MSG_SCALE * msg) * vm


def _conv1_body(ps_ref, pt_ref, zs_ref, emb_ref, w1_ref, b1_ref, r_ref,
                t_ref, w2z_ref, b2m_ref, msg_ref, rb_ref, vm_ref):
    rb, vm = _bessel(ps_ref[...], pt_ref[...])
    rb_ref[...] = rb
    vm_ref[...] = vm
    ids = lax.broadcasted_iota(jnp.int32, (1, 16), 1).astype(jnp.float32)
    oh = (jnp.abs(zs_ref[...] - ids) < 0.5).astype(jnp.float32)
    xin = jnp.dot(oh, emb_ref[...], preferred_element_type=jnp.float32)
    msg_ref[...] = _conv_math(rb, vm, xin, w1_ref, b1_ref,
                              r_ref, t_ref, w2z_ref, b2m_ref)


def _conv_body(rb_ref, vm_ref, xj_ref, w1_ref, b1_ref, r_ref, t_ref,
               w2z_ref, b2m_ref, msg_ref):
    xin = xj_ref[...]
    xin = _SILU_2MOM * xin * jax.nn.sigmoid(xin)
    msg_ref[...] = _conv_math(rb_ref[...], vm_ref[...], xin, w1_ref, b1_ref,
                              r_ref, t_ref, w2z_ref, b2m_ref)


def _wspecs():
    full = lambda a, b: pl.BlockSpec((a, b), lambda i: (0, 0))
    return [full(_NB, _H), full(1, _H), full(_H, _H * _D),
            full(_D, _H * _D), full(_H * _D, _D), full(_D, _D)]


def _conv1(psrc, ptgt, zsrc, emb16, *w):
    grid = _E_PAD // _BE
    return pl.pallas_call(
        _conv1_body,
        grid=(grid,),
        in_specs=[pl.BlockSpec((_BE, 16), lambda i: (i, 0)),
                  pl.BlockSpec((_BE, 16), lambda i: (i, 0)),
                  pl.BlockSpec((_BE, 1), lambda i: (i, 0)),
                  pl.BlockSpec((16, _D), lambda i: (0, 0))] + _wspecs(),
        out_specs=[pl.BlockSpec((_BE, _D), lambda i: (i, 0)),
                   pl.BlockSpec((_BE, _NB), lambda i: (i, 0)),
                   pl.BlockSpec((_BE, 1), lambda i: (i, 0))],
        out_shape=[jax.ShapeDtypeStruct((_E_PAD, _D), jnp.float32),
                   jax.ShapeDtypeStruct((_E_PAD, _NB), jnp.float32),
                   jax.ShapeDtypeStruct((_E_PAD, 1), jnp.float32)],
    )(psrc, ptgt, zsrc, emb16, *w)


def _conv(rb, vm, xj, *w):
    grid = _E_PAD // _BE
    return pl.pallas_call(
        _conv_body,
        grid=(grid,),
        in_specs=[pl.BlockSpec((_BE, _NB), lambda i: (i, 0)),
                  pl.BlockSpec((_BE, 1), lambda i: (i, 0)),
                  pl.BlockSpec((_BE, _D), lambda i: (i, 0))] + _wspecs(),
        out_specs=pl.BlockSpec((_BE, _D), lambda i: (i, 0)),
        out_shape=jax.ShapeDtypeStruct((_E_PAD, _D), jnp.float32),
    )(rb, vm, xj, *w)


def _expansion_mats():
    c = np.arange(_H * _D)
    r_mat = (c[None, :] // _D == np.arange(_H)[:, None]).astype(np.float32)
    t_mat = (c[None, :] % _D == np.arange(_D)[:, None]).astype(np.float32)
    return jnp.asarray(r_mat), jnp.asarray(t_mat)


def _pack_gidx(idx, nch, ch):
    n = _NW * nch * ch
    idx = jnp.pad(idx.astype(jnp.int32), (0, n - idx.shape[0]))
    return idx.reshape(_NW, nch, ch)


def _pack_sidx(idx, half, trash):
    idx = jnp.pad(idx.astype(jnp.int32), (0, _E_PAD - idx.shape[0]))
    cores = []
    for c in range(2):
        lo = c * half
        inh = (idx >= lo) & (idx < lo + half)
        cores.append(jnp.where(inh, idx - lo, trash).reshape(16, _SNCH, _ECH))
    return jnp.concatenate(cores, axis=0)


def kernel(edge_index, z, pos, batch, emb,
           w1_0, b1_0, w2_0, b2_0,
           w1_1, b1_1, w2_1, b2_1,
           w1_2, b1_2, w2_2, b2_2):
    src, tgt = edge_index[0], edge_index[1]
    src3 = _pack_gidx(src, _ENCH, _ECH)
    tgt3 = _pack_gidx(tgt, _ENCH, _ECH)
    zsrc = jnp.pad(jnp.take(z, src).astype(jnp.float32),
                   (0, _E_PAD - src.shape[0])).reshape(_E_PAD, 1)
    emb16 = jnp.pad(emb, ((0, 16 - emb.shape[0]), (0, 0)))
    srcb3 = _pack_gidx(src + 120 * (src >= _NHALF).astype(src.dtype),
                       _ENCH, _ECH)
    tgt_s = _pack_sidx(tgt, _NHALF, _NHALF)
    bg_s = _pack_sidx(jnp.take(batch, tgt), _NG // 2, _NG // 2)
    zeros = jnp.zeros((_NH, _D), jnp.float32)

    pos16 = jnp.pad(pos, ((0, 0), (0, 13)))
    psrc, ptgt = _sc_gather0(pos16, src3, tgt3)

    r_mat, t_mat = _expansion_mats()
    ws = [(w1_0, b1_0, w2_0, b2_0),
          (w1_1, b1_1, w2_1, b2_1),
          (w1_2, b1_2, w2_2, b2_2)]
    wargs = [(w1, b1.reshape(1, _H), r_mat, t_mat,
              w2.reshape(_H * _D, _D), b2.reshape(_D, _D))
             for (w1, b1, w2, b2) in ws]

    msg, rb, vm = _conv1(psrc, ptgt, zsrc, emb16, *wargs[0])
    x = _sc_scatter_add(msg, tgt_s, zeros, _NH, _NH)

    xj = _sc_gather(x, srcb3)
    msg = _conv(rb, vm, xj, *wargs[1])
    x = _sc_scatter_add(msg, tgt_s, zeros, _NH, _NH)

    xj = _sc_gather(x, srcb3)
    msg = _conv(rb, vm, xj, *wargs[2])
    return _sc_scatter_add(msg, bg_s, zeros, _NG, _NG // 2)


# speedup vs baseline: 1.6470x; 1.6470x over previous
import functools
import numpy as np
import jax
import jax.numpy as jnp
from jax import lax
from jax.experimental import pallas as pl
from jax.experimental.pallas import tpu as pltpu
from jax.experimental.pallas import tpu_sc as plsc

_N_NODES = 10000
_D = 32
_NB = 10
_H = 16
_CUTOFF = 4.0
_NG = 64
_SH0 = 0.28209479177387814
_SILU_2MOM = 1.6790590286254883
_PATH_W = 1.0 / float(np.sqrt(_D))
_MSG_SCALE = _SH0 * _PATH_W

_NW = 32
_E_PAD = 163840
_ECH = 128
_ENCH = _E_PAD // (_NW * _ECH)
_SNCH = _E_PAD // (16 * _ECH)
_N_PAD = 10240
_ZCH = 64
_ZNCH = _N_PAD // (_NW * _ZCH)
_NHALF = 5000
_NH = 5120
_BE = 2048

_SC_PARAMS = pltpu.CompilerParams(use_tc_tiling_on_sc=False)


def _sc_mesh():
    return plsc.VectorSubcoreMesh(core_axis_name="c", subcore_axis_name="s")


_GRP = 4


def _sc_gather0(pos16, zf16, btf16, src3, tgt3):
    @functools.partial(
        pl.kernel,
        out_type=tuple(jax.ShapeDtypeStruct((_E_PAD, 16), jnp.float32)
                       for _ in range(4)),
        mesh=_sc_mesh(),
        compiler_params=_SC_PARAMS,
        scratch_types=[
            pltpu.VMEM((_ENCH, _ECH), jnp.int32),
            pltpu.VMEM((_GRP * _ECH, 16), jnp.float32),
            pltpu.SemaphoreType.DMA,
        ],
    )
    def k(pos_hbm, z_hbm, bt_hbm, src_hbm, tgt_hbm,
          ps_out, pt_out, zs_out, bt_out, eidx, ebuf, sem):
        wid = lax.axis_index("s") * 2 + lax.axis_index("c")
        base = wid * (_ENCH * _ECH)
        for idx_hbm, tabs in ((src_hbm, ((pos_hbm, ps_out),
                                         (z_hbm, zs_out))),
                              (tgt_hbm, ((pos_hbm, pt_out),
                                         (bt_hbm, bt_out)))):
            pltpu.sync_copy(idx_hbm.at[wid], eidx)
            for tab, out in tabs:
                def body(g, carry, tab=tab, out=out):
                    cps = [
                        pltpu.async_copy(
                            tab.at[eidx.at[g * _GRP + j]],
                            ebuf.at[pl.ds(j * _ECH, _ECH)], sem)
                        for j in range(_GRP)
                    ]
                    for cp in cps:
                        cp.wait()
                    pltpu.sync_copy(
                        ebuf, out.at[pl.ds(base + g * (_GRP * _ECH),
                                           _GRP * _ECH)])
                    return carry
                lax.fori_loop(0, _ENCH // _GRP, body, 0)

    return k(pos16, zf16, btf16, src3, tgt3)


def _sc_gather(table, idx3):
    nw, nch, ch = idx3.shape
    out_rows = nw * nch * ch

    @functools.partial(
        pl.kernel,
        out_type=jax.ShapeDtypeStruct((out_rows, _D), jnp.float32),
        mesh=_sc_mesh(),
        compiler_params=_SC_PARAMS,
        scratch_types=[
            pltpu.VMEM((nch, ch), jnp.int32),
            pltpu.VMEM((_GRP * ch, _D), jnp.float32),
            pltpu.SemaphoreType.DMA,
        ],
    )
    def k(table_hbm, idx_hbm, out_hbm, idx_v, buf, sem):
        wid = lax.axis_index("s") * 2 + lax.axis_index("c")
        base = wid * (nch * ch)
        pltpu.sync_copy(idx_hbm.at[wid], idx_v)

        def body(g, carry):
            cps = [
                pltpu.async_copy(table_hbm.at[idx_v.at[g * _GRP + j]],
                                 buf.at[pl.ds(j * ch, ch)], sem)
                for j in range(_GRP)
            ]
            for cp in cps:
                cp.wait()
            pltpu.sync_copy(buf, out_hbm.at[pl.ds(base + g * (_GRP * ch),
                                                  _GRP * ch)])
            return carry
        lax.fori_loop(0, nch // _GRP, body, 0)

    return k(table, idx3)


def _sc_scatter_add(msgs, idx3, zeros, nh, nreal):
    zrows = nh // 16
    wrows = nreal // 16

    @functools.partial(
        pl.kernel,
        out_type=jax.ShapeDtypeStruct((2 * nreal, _D), jnp.float32),
        mesh=_sc_mesh(),
        compiler_params=_SC_PARAMS,
        scratch_types=[
            pltpu.VMEM((_SNCH, _ECH), jnp.int32),
            pltpu.VMEM((_GRP * _ECH, _D), jnp.float32),
            pltpu.VMEM_SHARED((nh, _D), jnp.float32),
            pltpu.SemaphoreType.DMA,
        ],
    )
    def k(msgs_hbm, idx_hbm, zeros_hbm, out_hbm, idx_v, buf, acc, sem):
        cid = lax.axis_index("c")
        sid = lax.axis_index("s")
        pltpu.sync_copy(zeros_hbm.at[pl.ds(sid * zrows, zrows)],
                        acc.at[pl.ds(sid * zrows, zrows)])
        plsc.subcore_barrier()
        pltpu.sync_copy(idx_hbm.at[cid * 16 + sid], idx_v)

        def body(g, carry):
            rows = (sid * _SNCH + g * _GRP) * _ECH
            pltpu.async_copy(msgs_hbm.at[pl.ds(rows, _GRP * _ECH)],
                             buf, sem).wait()
            for j in range(_GRP):
                pltpu.sync_copy(buf.at[pl.ds(j * _ECH, _ECH)],
                                acc.at[idx_v.at[g * _GRP + j]], add=True)
            return carry
        lax.fori_loop(0, _SNCH // _GRP, body, 0)
        plsc.subcore_barrier()
        pltpu.sync_copy(acc.at[pl.ds(sid * wrows, wrows)],
                        out_hbm.at[pl.ds(cid * nreal + sid * wrows, wrows)])

    return k(msgs, idx3, zeros)


def _bessel(ps, pt):
    rel = pt - ps
    d2 = jnp.sum(rel * rel, axis=1, keepdims=True)
    valid = (d2 > 0.0).astype(jnp.float32)
    xx = jnp.sqrt(d2) * (1.0 / _CUTOFF)
    inv = jnp.where(xx > 0.0, 1.0 / jnp.maximum(xx, 1e-30), 0.0)
    x5 = xx * xx * xx * xx * xx
    env = (inv - 28.0 * x5 + 48.0 * x5 * xx - 21.0 * x5 * xx * xx)
    env = env * (xx < 1.0).astype(jnp.float32)
    freq = np.pi * (lax.broadcasted_iota(jnp.int32, (1, _NB), 1)
                    .astype(jnp.float32) + 1.0)
    return env * jnp.sin(freq * xx), valid


def _conv_math(rb, vm, xin, w1_ref, b1_ref, r_ref, t_ref, w2z_ref, b2m_ref):
    h = jnp.dot(rb, w1_ref[...],
                preferred_element_type=jnp.float32) + b1_ref[...]
    h = h * jax.nn.sigmoid(h)
    hz = jnp.dot(h, r_ref[...], preferred_element_type=jnp.float32)
    xz = jnp.dot(xin, t_ref[...], preferred_element_type=jnp.float32)
    msg = jnp.dot(hz * xz, w2z_ref[...], preferred_element_type=jnp.float32)
    msg = msg + jnp.dot(xin, b2m_ref[...], preferred_element_type=jnp.float32)
    return (_MSG_SCALE * msg) * vm


def _conv1_body(ps_ref, pt_ref, zs_ref, emb_ref, w1_ref, b1_ref, r_ref,
                t_ref, w2z_ref, b2m_ref, msg_ref, rb_ref, vm_ref):
    rb, vm = _bessel(ps_ref[...], pt_ref[...])
    rb_ref[...] = rb
    vm_ref[...] = vm
    ids = lax.broadcasted_iota(jnp.int32, (1, 16), 1).astype(jnp.float32)
    oh = (jnp.abs(zs_ref[:, 0:1] - ids) < 0.5).astype(jnp.float32)
    xin = jnp.dot(oh, emb_ref[...], preferred_element_type=jnp.float32)
    msg_ref[...] = _conv_math(rb, vm, xin, w1_ref, b1_ref,
                              r_ref, t_ref, w2z_ref, b2m_ref)


def _conv_body(rb_ref, vm_ref, xj_ref, w1_ref, b1_ref, r_ref, t_ref,
               w2z_ref, b2m_ref, msg_ref):
    xin = xj_ref[...]
    xin = _SILU_2MOM * xin * jax.nn.sigmoid(xin)
    msg_ref[...] = _conv_math(rb_ref[...], vm_ref[...], xin, w1_ref, b1_ref,
                              r_ref, t_ref, w2z_ref, b2m_ref)


def _wspecs():
    full = lambda a, b: pl.BlockSpec((a, b), lambda i: (0, 0))
    return [full(_NB, _H), full(1, _H), full(_H, _H * _D),
            full(_D, _H * _D), full(_H * _D, _D), full(_D, _D)]


def _conv1(psrc, ptgt, zsrc, emb16, *w):
    grid = _E_PAD // _BE
    return pl.pallas_call(
        _conv1_body,
        grid=(grid,),
        in_specs=[pl.BlockSpec((_BE, 16), lambda i: (i, 0)),
                  pl.BlockSpec((_BE, 16), lambda i: (i, 0)),
                  pl.BlockSpec((_BE, 16), lambda i: (i, 0)),
                  pl.BlockSpec((16, _D), lambda i: (0, 0))] + _wspecs(),
        out_specs=[pl.BlockSpec((_BE, _D), lambda i: (i, 0)),
                   pl.BlockSpec((_BE, _NB), lambda i: (i, 0)),
                   pl.BlockSpec((_BE, 1), lambda i: (i, 0))],
        out_shape=[jax.ShapeDtypeStruct((_E_PAD, _D), jnp.float32),
                   jax.ShapeDtypeStruct((_E_PAD, _NB), jnp.float32),
                   jax.ShapeDtypeStruct((_E_PAD, 1), jnp.float32)],
    )(psrc, ptgt, zsrc, emb16, *w)


def _conv(rb, vm, xj, *w):
    grid = _E_PAD // _BE
    return pl.pallas_call(
        _conv_body,
        grid=(grid,),
        in_specs=[pl.BlockSpec((_BE, _NB), lambda i: (i, 0)),
                  pl.BlockSpec((_BE, 1), lambda i: (i, 0)),
                  pl.BlockSpec((_BE, _D), lambda i: (i, 0))] + _wspecs(),
        out_specs=pl.BlockSpec((_BE, _D), lambda i: (i, 0)),
        out_shape=jax.ShapeDtypeStruct((_E_PAD, _D), jnp.float32),
    )(rb, vm, xj, *w)


def _expansion_mats():
    c = np.arange(_H * _D)
    r_mat = (c[None, :] // _D == np.arange(_H)[:, None]).astype(np.float32)
    t_mat = (c[None, :] % _D == np.arange(_D)[:, None]).astype(np.float32)
    return jnp.asarray(r_mat), jnp.asarray(t_mat)


def _pack_gidx(idx, nch, ch):
    n = _NW * nch * ch
    idx = jnp.pad(idx.astype(jnp.int32), (0, n - idx.shape[0]))
    return idx.reshape(_NW, nch, ch)


def _pack_sidx(idx, half, trash):
    cores = []
    for c in range(2):
        lo = c * half
        inh = (idx >= lo) & (idx < lo + half)
        cores.append(jnp.where(inh, idx - lo, trash).reshape(16, _SNCH, _ECH))
    return jnp.concatenate(cores, axis=0)


def kernel(edge_index, z, pos, batch, emb,
           w1_0, b1_0, w2_0, b2_0,
           w1_1, b1_1, w2_1, b2_1,
           w1_2, b1_2, w2_2, b2_2):
    src, tgt = edge_index[0], edge_index[1]
    src3 = _pack_gidx(src, _ENCH, _ECH)
    tgt3 = _pack_gidx(tgt, _ENCH, _ECH)
    emb16 = jnp.pad(emb, ((0, 16 - emb.shape[0]), (0, 0)))
    srcb3 = _pack_gidx(src + 120 * (src >= _NHALF).astype(src.dtype),
                       _ENCH, _ECH)
    tgt_pad = jnp.pad(tgt.astype(jnp.int32), (0, _E_PAD - tgt.shape[0]))
    tgt_s = _pack_sidx(tgt_pad, _NHALF, _NHALF)
    zeros = jnp.zeros((_NH, _D), jnp.float32)

    pos16 = jnp.pad(pos, ((0, 0), (0, 13)))
    zf16 = jnp.broadcast_to(z.astype(jnp.float32)[:, None], (_N_NODES, 16))
    btf16 = jnp.broadcast_to(batch.astype(jnp.float32)[:, None],
                             (_N_NODES, 16))
    psrc, ptgt, zs16, bt16 = _sc_gather0(pos16, zf16, btf16, src3, tgt3)
    bg_s = _pack_sidx(bt16[:, 0].astype(jnp.int32), _NG // 2, _NG // 2)

    r_mat, t_mat = _expansion_mats()
    ws = [(w1_0, b1_0, w2_0, b2_0),
          (w1_1, b1_1, w2_1, b2_1),
          (w1_2, b1_2, w2_2, b2_2)]
    wargs = [(w1, b1.reshape(1, _H), r_mat, t_mat,
              w2.reshape(_H * _D, _D), b2.reshape(_D, _D))
             for (w1, b1, w2, b2) in ws]

    msg, rb, vm = _conv1(psrc, ptgt, zs16, emb16, *wargs[0])
    x = _sc_scatter_add(msg, tgt_s, zeros, _NH, _NH)

    xj = _sc_gather(x, srcb3)
    msg = _conv(rb, vm, xj, *wargs[1])
    x = _sc_scatter_add(msg, tgt_s, zeros, _NH, _NH)

    xj = _sc_gather(x, srcb3)
    msg = _conv(rb, vm, xj, *wargs[2])
    return _sc_scatter_add(msg, bg_s, zeros, _NG, _NG // 2)


# speedup vs baseline: 1.6540x; 1.0043x over previous
import functools
import numpy as np
import jax
import jax.numpy as jnp
from jax import lax
from jax.experimental import pallas as pl
from jax.experimental.pallas import tpu as pltpu
from jax.experimental.pallas import tpu_sc as plsc

_N_NODES = 10000
_D = 32
_NB = 10
_H = 16
_CUTOFF = 4.0
_NG = 64
_SH0 = 0.28209479177387814
_SILU_2MOM = 1.6790590286254883
_PATH_W = 1.0 / float(np.sqrt(_D))
_MSG_SCALE = _SH0 * _PATH_W

_NW = 32
_E_PAD = 163840
_ECH = 128
_ENCH = _E_PAD // (_NW * _ECH)
_SNCH = _E_PAD // (16 * _ECH)
_N_PAD = 10240
_ZCH = 64
_ZNCH = _N_PAD // (_NW * _ZCH)
_NHALF = 5000
_NH = 5120
_BE = 2048

_SC_PARAMS = pltpu.CompilerParams(use_tc_tiling_on_sc=False)


def _sc_mesh():
    return plsc.VectorSubcoreMesh(core_axis_name="c", subcore_axis_name="s")


_GRP = 8


def _sc_gather0(pos16, zf16, btf16, src3, tgt3):
    @functools.partial(
        pl.kernel,
        out_type=tuple(jax.ShapeDtypeStruct((_E_PAD, 16), jnp.float32)
                       for _ in range(4)),
        mesh=_sc_mesh(),
        compiler_params=_SC_PARAMS,
        scratch_types=[
            pltpu.VMEM((_ENCH, _ECH), jnp.int32),
            pltpu.VMEM((_GRP * _ECH, 16), jnp.float32),
            pltpu.SemaphoreType.DMA,
        ],
    )
    def k(pos_hbm, z_hbm, bt_hbm, src_hbm, tgt_hbm,
          ps_out, pt_out, zs_out, bt_out, eidx, ebuf, sem):
        wid = lax.axis_index("s") * 2 + lax.axis_index("c")
        base = wid * (_ENCH * _ECH)
        for idx_hbm, tabs in ((src_hbm, ((pos_hbm, ps_out),
                                         (z_hbm, zs_out))),
                              (tgt_hbm, ((pos_hbm, pt_out),
                                         (bt_hbm, bt_out)))):
            pltpu.sync_copy(idx_hbm.at[wid], eidx)
            for tab, out in tabs:
                def body(g, carry, tab=tab, out=out):
                    cps = [
                        pltpu.async_copy(
                            tab.at[eidx.at[g * _GRP + j]],
                            ebuf.at[pl.ds(j * _ECH, _ECH)], sem)
                        for j in range(_GRP)
                    ]
                    for cp in cps:
                        cp.wait()
                    pltpu.sync_copy(
                        ebuf, out.at[pl.ds(base + g * (_GRP * _ECH),
                                           _GRP * _ECH)])
                    return carry
                lax.fori_loop(0, _ENCH // _GRP, body, 0)

    return k(pos16, zf16, btf16, src3, tgt3)


def _sc_gather(table, idx3):
    nw, nch, ch = idx3.shape
    out_rows = nw * nch * ch

    @functools.partial(
        pl.kernel,
        out_type=jax.ShapeDtypeStruct((out_rows, _D), jnp.float32),
        mesh=_sc_mesh(),
        compiler_params=_SC_PARAMS,
        scratch_types=[
            pltpu.VMEM((nch, ch), jnp.int32),
            pltpu.VMEM((_GRP * ch, _D), jnp.float32),
            pltpu.SemaphoreType.DMA,
        ],
    )
    def k(table_hbm, idx_hbm, out_hbm, idx_v, buf, sem):
        wid = lax.axis_index("s") * 2 + lax.axis_index("c")
        base = wid * (nch * ch)
        pltpu.sync_copy(idx_hbm.at[wid], idx_v)

        def body(g, carry):
            cps = [
                pltpu.async_copy(table_hbm.at[idx_v.at[g * _GRP + j]],
                                 buf.at[pl.ds(j * ch, ch)], sem)
                for j in range(_GRP)
            ]
            for cp in cps:
                cp.wait()
            pltpu.sync_copy(buf, out_hbm.at[pl.ds(base + g * (_GRP * ch),
                                                  _GRP * ch)])
            return carry
        lax.fori_loop(0, nch // _GRP, body, 0)

    return k(table, idx3)


def _sc_scatter_add(msgs, idx3, zeros, nh, nreal):
    zrows = nh // 16
    wrows = nreal // 16

    @functools.partial(
        pl.kernel,
        out_type=jax.ShapeDtypeStruct((2 * nreal, _D), jnp.float32),
        mesh=_sc_mesh(),
        compiler_params=_SC_PARAMS,
        scratch_types=[
            pltpu.VMEM((_SNCH, _ECH), jnp.int32),
            pltpu.VMEM((_GRP * _ECH, _D), jnp.float32),
            pltpu.VMEM_SHARED((nh, _D), jnp.float32),
            pltpu.SemaphoreType.DMA,
        ],
    )
    def k(msgs_hbm, idx_hbm, zeros_hbm, out_hbm, idx_v, buf, acc, sem):
        cid = lax.axis_index("c")
        sid = lax.axis_index("s")
        pltpu.sync_copy(zeros_hbm.at[pl.ds(sid * zrows, zrows)],
                        acc.at[pl.ds(sid * zrows, zrows)])
        plsc.subcore_barrier()
        pltpu.sync_copy(idx_hbm.at[cid * 16 + sid], idx_v)

        def body(g, carry):
            rows = (sid * _SNCH + g * _GRP) * _ECH
            pltpu.async_copy(msgs_hbm.at[pl.ds(rows, _GRP * _ECH)],
                             buf, sem).wait()
            for j in range(_GRP):
                pltpu.sync_copy(buf.at[pl.ds(j * _ECH, _ECH)],
                                acc.at[idx_v.at[g * _GRP + j]], add=True)
            return carry
        lax.fori_loop(0, _SNCH // _GRP, body, 0)
        plsc.subcore_barrier()
        pltpu.sync_copy(acc.at[pl.ds(sid * wrows, wrows)],
                        out_hbm.at[pl.ds(cid * nreal + sid * wrows, wrows)])

    return k(msgs, idx3, zeros)


def _bessel(ps, pt):
    rel = pt - ps
    d2 = jnp.sum(rel * rel, axis=1, keepdims=True)
    valid = (d2 > 0.0).astype(jnp.float32)
    xx = jnp.sqrt(d2) * (1.0 / _CUTOFF)
    inv = jnp.where(xx > 0.0, 1.0 / jnp.maximum(xx, 1e-30), 0.0)
    x5 = xx * xx * xx * xx * xx
    env = (inv - 28.0 * x5 + 48.0 * x5 * xx - 21.0 * x5 * xx * xx)
    env = env * (xx < 1.0).astype(jnp.float32)
    freq = np.pi * (lax.broadcasted_iota(jnp.int32, (1, _NB), 1)
                    .astype(jnp.float32) + 1.0)
    return env * jnp.sin(freq * xx), valid


def _conv_math(rb, vm, xin, w1_ref, b1_ref, r_ref, t_ref, w2z_ref, b2m_ref):
    h = jnp.dot(rb, w1_ref[...],
                preferred_element_type=jnp.float32) + b1_ref[...]
    h = h * jax.nn.sigmoid(h)
    hz = jnp.dot(h, r_ref[...], preferred_element_type=jnp.float32)
    xz = jnp.dot(xin, t_ref[...], preferred_element_type=jnp.float32)
    msg = jnp.dot(hz * xz, w2z_ref[...], preferred_element_type=jnp.float32)
    msg = msg + jnp.dot(xin, b2m_ref[...], preferred_element_type=jnp.float32)
    return (_MSG_SCALE * msg) * vm


def _conv1_body(ps_ref, pt_ref, zs_ref, emb_ref, w1_ref, b1_ref, r_ref,
                t_ref, w2z_ref, b2m_ref, msg_ref, rb_ref, vm_ref):
    rb, vm = _bessel(ps_ref[...], pt_ref[...])
    rb_ref[...] = rb
    vm_ref[...] = vm
    ids = lax.broadcasted_iota(jnp.int32, (1, 16), 1).astype(jnp.float32)
    oh = (jnp.abs(zs_ref[:, 0:1] - ids) < 0.5).astype(jnp.float32)
    xin = jnp.dot(oh, emb_ref[...], preferred_element_type=jnp.float32)
    msg_ref[...] = _conv_math(rb, vm, xin, w1_ref, b1_ref,
                              r_ref, t_ref, w2z_ref, b2m_ref)


def _conv_body(rb_ref, vm_ref, xj_ref, w1_ref, b1_ref, r_ref, t_ref,
               w2z_ref, b2m_ref, msg_ref):
    xin = xj_ref[...]
    xin = _SILU_2MOM * xin * jax.nn.sigmoid(xin)
    msg_ref[...] = _conv_math(rb_ref[...], vm_ref[...], xin, w1_ref, b1_ref,
                              r_ref, t_ref, w2z_ref, b2m_ref)


def _wspecs():
    full = lambda a, b: pl.BlockSpec((a, b), lambda i: (0, 0))
    return [full(_NB, _H), full(1, _H), full(_H, _H * _D),
            full(_D, _H * _D), full(_H * _D, _D), full(_D, _D)]


def _conv1(psrc, ptgt, zsrc, emb16, *w):
    grid = _E_PAD // _BE
    return pl.pallas_call(
        _conv1_body,
        grid=(grid,),
        in_specs=[pl.BlockSpec((_BE, 16), lambda i: (i, 0)),
                  pl.BlockSpec((_BE, 16), lambda i: (i, 0)),
                  pl.BlockSpec((_BE, 16), lambda i: (i, 0)),
                  pl.BlockSpec((16, _D), lambda i: (0, 0))] + _wspecs(),
        out_specs=[pl.BlockSpec((_BE, _D), lambda i: (i, 0)),
                   pl.BlockSpec((_BE, _NB), lambda i: (i, 0)),
                   pl.BlockSpec((_BE, 1), lambda i: (i, 0))],
        out_shape=[jax.ShapeDtypeStruct((_E_PAD, _D), jnp.float32),
                   jax.ShapeDtypeStruct((_E_PAD, _NB), jnp.float32),
                   jax.ShapeDtypeStruct((_E_PAD, 1), jnp.float32)],
    )(psrc, ptgt, zsrc, emb16, *w)


def _conv(rb, vm, xj, *w):
    grid = _E_PAD // _BE
    return pl.pallas_call(
        _conv_body,
        grid=(grid,),
        in_specs=[pl.BlockSpec((_BE, _NB), lambda i: (i, 0)),
                  pl.BlockSpec((_BE, 1), lambda i: (i, 0)),
                  pl.BlockSpec((_BE, _D), lambda i: (i, 0))] + _wspecs(),
        out_specs=pl.BlockSpec((_BE, _D), lambda i: (i, 0)),
        out_shape=jax.ShapeDtypeStruct((_E_PAD, _D), jnp.float32),
    )(rb, vm, xj, *w)


def _expansion_mats():
    c = np.arange(_H * _D)
    r_mat = (c[None, :] // _D == np.arange(_H)[:, None]).astype(np.float32)
    t_mat = (c[None, :] % _D == np.arange(_D)[:, None]).astype(np.float32)
    return jnp.asarray(r_mat), jnp.asarray(t_mat)


def _pack_gidx(idx, nch, ch):
    n = _NW * nch * ch
    idx = jnp.pad(idx.astype(jnp.int32), (0, n - idx.shape[0]))
    return idx.reshape(_NW, nch, ch)


def _pack_sidx(idx, half, trash):
    cores = []
    for c in range(2):
        lo = c * half
        inh = (idx >= lo) & (idx < lo + half)
        cores.append(jnp.where(inh, idx - lo, trash).reshape(16, _SNCH, _ECH))
    return jnp.concatenate(cores, axis=0)


def kernel(edge_index, z, pos, batch, emb,
           w1_0, b1_0, w2_0, b2_0,
           w1_1, b1_1, w2_1, b2_1,
           w1_2, b1_2, w2_2, b2_2):
    src, tgt = edge_index[0], edge_index[1]
    src3 = _pack_gidx(src, _ENCH, _ECH)
    tgt3 = _pack_gidx(tgt, _ENCH, _ECH)
    emb16 = jnp.pad(emb, ((0, 16 - emb.shape[0]), (0, 0)))
    srcb3 = _pack_gidx(src + 120 * (src >= _NHALF).astype(src.dtype),
                       _ENCH, _ECH)
    tgt_pad = jnp.pad(tgt.astype(jnp.int32), (0, _E_PAD - tgt.shape[0]))
    tgt_s = _pack_sidx(tgt_pad, _NHALF, _NHALF)
    zeros = jnp.zeros((_NH, _D), jnp.float32)

    pos16 = jnp.pad(pos, ((0, 0), (0, 13)))
    zf16 = jnp.broadcast_to(z.astype(jnp.float32)[:, None], (_N_NODES, 16))
    btf16 = jnp.broadcast_to(batch.astype(jnp.float32)[:, None],
                             (_N_NODES, 16))
    psrc, ptgt, zs16, bt16 = _sc_gather0(pos16, zf16, btf16, src3, tgt3)
    bg_s = _pack_sidx(bt16[:, 0].astype(jnp.int32), _NG // 2, _NG // 2)

    r_mat, t_mat = _expansion_mats()
    ws = [(w1_0, b1_0, w2_0, b2_0),
          (w1_1, b1_1, w2_1, b2_1),
          (w1_2, b1_2, w2_2, b2_2)]
    wargs = [(w1, b1.reshape(1, _H), r_mat, t_mat,
              w2.reshape(_H * _D, _D), b2.reshape(_D, _D))
             for (w1, b1, w2, b2) in ws]

    msg, rb, vm = _conv1(psrc, ptgt, zs16, emb16, *wargs[0])
    x = _sc_scatter_add(msg, tgt_s, zeros, _NH, _NH)

    xj = _sc_gather(x, srcb3)
    msg = _conv(rb, vm, xj, *wargs[1])
    x = _sc_scatter_add(msg, tgt_s, zeros, _NH, _NH)

    xj = _sc_gather(x, srcb3)
    msg = _conv(rb, vm, xj, *wargs[2])
    return _sc_scatter_add(msg, bg_s, zeros, _NG, _NG // 2)
